# Initial kernel scaffold; baseline (speedup 1.0000x reference)
import jax
import jax.numpy as jnp
from jax.experimental import pallas as pl


def kernel(rays_o, rays_d, binaries):
    raise NotImplementedError("write your pallas kernel here")


# speedup vs baseline: 198.6166x; 198.6166x over previous
import functools
import math

import jax
import jax.numpy as jnp
from jax import lax
from jax.experimental import pallas as pl
from jax.experimental.pallas import tpu as pltpu
from jax.experimental.pallas import tpu_sc as plsc

RADIUS = 1.0
RES = 128
STEPS = 512
STEP = RADIUS * 2.0 * math.sqrt(3.0) / STEPS
NRAYS = 16384
L = 16
NC, NS = 2, 16
NW = NC * NS
RAYS_PER_W = NRAYS // NW
GROUPS = RAYS_PER_W // L
NWORDS = RES * RES * (RES // 32)

_mesh = plsc.VectorSubcoreMesh(core_axis_name="c", subcore_axis_name="s")


@functools.partial(
    pl.kernel,
    mesh=_mesh,
    out_type=[
        jax.ShapeDtypeStruct((NRAYS * STEPS,), jnp.int32),
        jax.ShapeDtypeStruct((NRAYS * STEPS,), jnp.float32),
        jax.ShapeDtypeStruct((NRAYS * STEPS,), jnp.float32),
    ],
    compiler_params=pltpu.CompilerParams(needs_layout_passes=False),
    scratch_types=[
        pltpu.VMEM((NWORDS,), jnp.int32),
        pltpu.VMEM((RAYS_PER_W,), jnp.float32),
        pltpu.VMEM((RAYS_PER_W,), jnp.float32),
        pltpu.VMEM((RAYS_PER_W,), jnp.float32),
        pltpu.VMEM((RAYS_PER_W,), jnp.float32),
        pltpu.VMEM((RAYS_PER_W,), jnp.float32),
        pltpu.VMEM((RAYS_PER_W,), jnp.float32),
        pltpu.VMEM((L,), jnp.float32),
        pltpu.VMEM((L * STEPS,), jnp.int32),
        pltpu.VMEM((L * STEPS,), jnp.float32),
        pltpu.VMEM((L * STEPS,), jnp.float32),
    ],
)
def _march(ox_h, oy_h, oz_h, dx_h, dy_h, dz_h, tab_h,
           ri_h, ts_h, te_h,
           tab_v, ox_v, oy_v, oz_v, dx_v, dy_v, dz_v,
           red_v, ri_v, ts_v, te_v):
    wid = lax.axis_index("s") * NC + lax.axis_index("c")
    ray_base = wid * RAYS_PER_W
    pltpu.sync_copy(tab_h, tab_v)
    pltpu.sync_copy(ox_h.at[pl.ds(ray_base, RAYS_PER_W)], ox_v)
    pltpu.sync_copy(oy_h.at[pl.ds(ray_base, RAYS_PER_W)], oy_v)
    pltpu.sync_copy(oz_h.at[pl.ds(ray_base, RAYS_PER_W)], oz_v)
    pltpu.sync_copy(dx_h.at[pl.ds(ray_base, RAYS_PER_W)], dx_v)
    pltpu.sync_copy(dy_h.at[pl.ds(ray_base, RAYS_PER_W)], dy_v)
    pltpu.sync_copy(dz_h.at[pl.ds(ray_base, RAYS_PER_W)], dz_v)

    iota = lax.iota(jnp.int32, L)
    oidx0 = iota * STEPS
    neg1 = jnp.full((L,), -1, jnp.int32)
    zero = jnp.zeros((L,), jnp.float32)

    def group_body(g, carry):
        o_x = ox_v[pl.ds(g * L, L)]
        o_y = oy_v[pl.ds(g * L, L)]
        o_z = oz_v[pl.ds(g * L, L)]
        d_x = dx_v[pl.ds(g * L, L)]
        d_y = dy_v[pl.ds(g * L, L)]
        d_z = dz_v[pl.ds(g * L, L)]

        def axis_ts(o_a, d_a):
            safe = jnp.where(jnp.abs(d_a) < 1e-10, 1e-10, d_a)
            inv = 1.0 / safe
            t0 = (-1.0 - o_a) * inv
            t1 = (1.0 - o_a) * inv
            return jnp.minimum(t0, t1), jnp.maximum(t0, t1)
        nx, xx = axis_ts(o_x, d_x)
        ny, xy = axis_ts(o_y, d_y)
        nz, xz = axis_ts(o_z, d_z)
        t_near = jnp.maximum(jnp.maximum(jnp.maximum(nx, ny), nz), 0.0)
        t_far = jnp.minimum(jnp.minimum(xx, xy), xz)
        tfar_eff = jnp.where(t_far > t_near, t_far, -jnp.inf)

        span = jnp.maximum(tfar_eff - t_near, 0.0)
        span_max = span[0]
        for lane in range(1, L):
            span_max = jnp.maximum(span_max, span[lane])
        nmax = jnp.minimum((span_max * (1.0 / STEP)).astype(jnp.int32) + 2,
                           STEPS)

        rid = (ray_base + g * L) + iota

        def step_body(i, fi):
            t_s = t_near + fi * STEP
            t_e = t_s + STEP
            t_mid = (t_s + t_e) * 0.5
            px = o_x + d_x * t_mid
            py = o_y + d_y * t_mid
            pz = o_z + d_z * t_mid
            cx = jnp.clip((px + 1.0) * 64.0, 0.0, 127.0).astype(jnp.int32)
            cy = jnp.clip((py + 1.0) * 64.0, 0.0, 127.0).astype(jnp.int32)
            cz = jnp.clip((pz + 1.0) * 64.0, 0.0, 127.0).astype(jnp.int32)
            widx = (cx << 9) | (cy << 2) | (cz >> 5)
            word = plsc.load_gather(tab_v, [widx])
            bit = jnp.right_shift(word, cz & 31) & 1
            sig = jnp.maximum(px, 0.0)
            alpha_pos = (1.0 - jnp.exp(sig * (-STEP))) > 0.0
            m = (t_e <= tfar_eff) & (bit != 0) & alpha_pos
            oidx = oidx0 + i
            plsc.store_scatter(ri_v, [oidx], jnp.where(m, rid, -1))
            plsc.store_scatter(ts_v, [oidx], jnp.where(m, t_s, 0.0))
            plsc.store_scatter(te_v, [oidx], jnp.where(m, t_e, 0.0))
            return fi + 1.0

        lax.fori_loop(0, nmax, step_body, jnp.zeros((L,), jnp.float32),
                      unroll=False)

        def fill_body(i, _):
            oidx = oidx0 + i
            plsc.store_scatter(ri_v, [oidx], neg1)
            plsc.store_scatter(ts_v, [oidx], zero)
            plsc.store_scatter(te_v, [oidx], zero)
            return 0
        lax.fori_loop(nmax, STEPS, fill_body, 0, unroll=False)

        out_base = (ray_base + g * L) * STEPS
        pltpu.sync_copy(ri_v, ri_h.at[pl.ds(out_base, L * STEPS)])
        pltpu.sync_copy(ts_v, ts_h.at[pl.ds(out_base, L * STEPS)])
        pltpu.sync_copy(te_v, te_h.at[pl.ds(out_base, L * STEPS)])
        return carry

    lax.fori_loop(0, GROUPS, group_body, 0, unroll=False)


def _pack_grid(binaries):
    b = binaries[0].astype(jnp.uint32)
    bm = b.reshape(RES, RES, RES // 32, 32)
    w = jnp.left_shift(jnp.uint32(1), jnp.arange(32, dtype=jnp.uint32))
    packed = jnp.sum(bm * w, axis=-1, dtype=jnp.uint32).reshape(-1)
    return lax.bitcast_convert_type(packed, jnp.int32)


def kernel(rays_o, rays_d, binaries):
    tab = _pack_grid(binaries)
    ox, oy, oz = rays_o[:, 0], rays_o[:, 1], rays_o[:, 2]
    dx, dy, dz = rays_d[:, 0], rays_d[:, 1], rays_d[:, 2]
    ri, ts, te = _march(ox, oy, oz, dx, dy, dz, tab)
    return ri, ts, te


# speedup vs baseline: 200.6500x; 1.0102x over previous
import functools
import math

import jax
import jax.numpy as jnp
from jax import lax
from jax.experimental import pallas as pl
from jax.experimental.pallas import tpu as pltpu
from jax.experimental.pallas import tpu_sc as plsc

RADIUS = 1.0
RES = 128
STEPS = 512
STEP = RADIUS * 2.0 * math.sqrt(3.0) / STEPS
NRAYS = 16384
L = 16
NC, NS = 2, 16
NW = NC * NS
RAYS_PER_W = NRAYS // NW
GROUPS = RAYS_PER_W // L
NWORDS = RES * RES * (RES // 32)
U = 4

_mesh = plsc.VectorSubcoreMesh(core_axis_name="c", subcore_axis_name="s")


@functools.partial(
    pl.kernel,
    mesh=_mesh,
    out_type=[
        jax.ShapeDtypeStruct((NRAYS * STEPS,), jnp.int32),
        jax.ShapeDtypeStruct((NRAYS * STEPS,), jnp.float32),
        jax.ShapeDtypeStruct((NRAYS * STEPS,), jnp.float32),
    ],
    compiler_params=pltpu.CompilerParams(needs_layout_passes=False),
    scratch_types=[
        pltpu.VMEM((NWORDS,), jnp.int32),
        pltpu.VMEM((RAYS_PER_W,), jnp.float32),
        pltpu.VMEM((RAYS_PER_W,), jnp.float32),
        pltpu.VMEM((RAYS_PER_W,), jnp.float32),
        pltpu.VMEM((RAYS_PER_W,), jnp.float32),
        pltpu.VMEM((RAYS_PER_W,), jnp.float32),
        pltpu.VMEM((RAYS_PER_W,), jnp.float32),
        pltpu.VMEM((L * STEPS,), jnp.int32),
        pltpu.VMEM((L * STEPS,), jnp.float32),
        pltpu.VMEM((L * STEPS,), jnp.float32),
    ],
)
def _march(ox_h, oy_h, oz_h, dx_h, dy_h, dz_h, tab_h,
           ri_h, ts_h, te_h,
           tab_v, ox_v, oy_v, oz_v, dx_v, dy_v, dz_v,
           ri_v, ts_v, te_v):
    wid = lax.axis_index("s") * NC + lax.axis_index("c")
    ray_base = wid * RAYS_PER_W
    pltpu.sync_copy(tab_h, tab_v)
    pltpu.sync_copy(ox_h.at[pl.ds(ray_base, RAYS_PER_W)], ox_v)
    pltpu.sync_copy(oy_h.at[pl.ds(ray_base, RAYS_PER_W)], oy_v)
    pltpu.sync_copy(oz_h.at[pl.ds(ray_base, RAYS_PER_W)], oz_v)
    pltpu.sync_copy(dx_h.at[pl.ds(ray_base, RAYS_PER_W)], dx_v)
    pltpu.sync_copy(dy_h.at[pl.ds(ray_base, RAYS_PER_W)], dy_v)
    pltpu.sync_copy(dz_h.at[pl.ds(ray_base, RAYS_PER_W)], dz_v)

    iota = lax.iota(jnp.int32, L)
    oidx0 = iota * STEPS
    neg1 = jnp.full((L,), -1, jnp.int32)
    zero = jnp.zeros((L,), jnp.float32)

    def group_body(g, carry):
        o_x = ox_v[pl.ds(g * L, L)]
        o_y = oy_v[pl.ds(g * L, L)]
        o_z = oz_v[pl.ds(g * L, L)]
        d_x = dx_v[pl.ds(g * L, L)]
        d_y = dy_v[pl.ds(g * L, L)]
        d_z = dz_v[pl.ds(g * L, L)]

        def axis_ts(o_a, d_a):
            safe = jnp.where(jnp.abs(d_a) < 1e-10, 1e-10, d_a)
            inv = 1.0 / safe
            t0 = (-1.0 - o_a) * inv
            t1 = (1.0 - o_a) * inv
            return jnp.minimum(t0, t1), jnp.maximum(t0, t1)
        nx, xx = axis_ts(o_x, d_x)
        ny, xy = axis_ts(o_y, d_y)
        nz, xz = axis_ts(o_z, d_z)
        t_near = jnp.maximum(jnp.maximum(jnp.maximum(nx, ny), nz), 0.0)
        t_far = jnp.minimum(jnp.minimum(xx, xy), xz)
        tfar_eff = jnp.where(t_far > t_near, t_far, -jnp.inf)

        span = jnp.maximum(tfar_eff - t_near, 0.0)
        span_max = span[0]
        for lane in range(1, L):
            span_max = jnp.maximum(span_max, span[lane])
        nmax = jnp.minimum((span_max * (1.0 / STEP)).astype(jnp.int32) + 2,
                           STEPS)
        cnt_u = jnp.minimum((nmax + (U - 1)) // U, STEPS // U)

        rid = (ray_base + g * L) + iota

        def one_step(i, fi):
            t_s = t_near + fi * STEP
            t_e = t_s + STEP
            t_mid = (t_s + t_e) * 0.5
            px = o_x + d_x * t_mid
            py = o_y + d_y * t_mid
            pz = o_z + d_z * t_mid
            cx = jnp.clip((px + 1.0) * 64.0, 0.0, 127.0).astype(jnp.int32)
            cy = jnp.clip((py + 1.0) * 64.0, 0.0, 127.0).astype(jnp.int32)
            cz = jnp.clip((pz + 1.0) * 64.0, 0.0, 127.0).astype(jnp.int32)
            widx = (cx << 9) | (cy << 2) | (cz >> 5)
            word = plsc.load_gather(tab_v, [widx])
            bit = jnp.right_shift(word, cz & 31) & 1
            sig = jnp.maximum(px, 0.0)
            alpha_pos = jnp.exp(sig * (-STEP)) < 1.0
            m = (t_e <= tfar_eff) & (bit != 0) & alpha_pos
            oidx = oidx0 + i
            plsc.store_scatter(ri_v, [oidx], jnp.where(m, rid, -1))
            plsc.store_scatter(ts_v, [oidx], jnp.where(m, t_s, 0.0))
            plsc.store_scatter(te_v, [oidx], jnp.where(m, t_e, 0.0))

        def step_body(iu, fi):
            base = iu * U
            for u in range(U):
                one_step(base + u, fi + float(u))
            return fi + float(U)

        lax.fori_loop(0, cnt_u, step_body, 0.0)

        def fill_body(iu, c):
            base = iu * U
            for u in range(U):
                oidx = oidx0 + (base + u)
                plsc.store_scatter(ri_v, [oidx], neg1)
                plsc.store_scatter(ts_v, [oidx], zero)
                plsc.store_scatter(te_v, [oidx], zero)
            return c
        lax.fori_loop(cnt_u, STEPS // U, fill_body, 0)

        out_base = (ray_base + g * L) * STEPS
        pltpu.sync_copy(ri_v, ri_h.at[pl.ds(out_base, L * STEPS)])
        pltpu.sync_copy(ts_v, ts_h.at[pl.ds(out_base, L * STEPS)])
        pltpu.sync_copy(te_v, te_h.at[pl.ds(out_base, L * STEPS)])
        return carry

    lax.fori_loop(0, GROUPS, group_body, 0, unroll=False)


def _pack_grid(binaries):
    b = binaries[0].astype(jnp.uint32)
    bm = b.reshape(RES, RES, RES // 32, 32)
    w = jnp.left_shift(jnp.uint32(1), jnp.arange(32, dtype=jnp.uint32))
    packed = jnp.sum(bm * w, axis=-1, dtype=jnp.uint32).reshape(-1)
    return lax.bitcast_convert_type(packed, jnp.int32)


def kernel(rays_o, rays_d, binaries):
    tab = _pack_grid(binaries)
    ox, oy, oz = rays_o[:, 0], rays_o[:, 1], rays_o[:, 2]
    dx, dy, dz = rays_d[:, 0], rays_d[:, 1], rays_d[:, 2]
    ri, ts, te = _march(ox, oy, oz, dx, dy, dz, tab)
    return ri, ts, te
